# baseline (speedup 1.0000x reference)
import functools

import jax
import jax.numpy as jnp
from jax import lax
from jax.experimental import pallas as pl
from jax.experimental.pallas import tpu as pltpu
from jax.experimental.pallas import tpu_sc as plsc

N = 10000
IN_DIM = 14
OUT_DIM = 8
K = 64
NP = 10240
DT = 32
PID_COL = 30
B = N * K

SC_CORES = 2
SC_SUBCORES = 16
NW = SC_CORES * SC_SUBCORES
CHUNK = 512
NCHUNKS = B // CHUNK
CHUNKS_PER_W = -(-NCHUNKS // NW)

QB1 = 512
QB2 = 256
QB4 = 40
EB4 = QB4 * K


def _rbf(a):
    u = jax.lax.bitcast_convert_type(a, jnp.uint32)
    r = (u + 0x7FFF + ((u >> 16) & 1)) & jnp.uint32(0xFFFF0000)
    return jax.lax.bitcast_convert_type(r, jnp.float32)


def _dot3(a, b):
    ah = _rbf(a)
    al = _rbf(a - ah)
    bh = _rbf(b)
    bl = _rbf(b - bh)
    d = functools.partial(jnp.dot, preferred_element_type=jnp.float32)
    return d(al, bh) + d(ah, bl) + d(ah, bh)


def _embed_body(x_ref, w1_ref, b1_ref, w2_ref, b2_ref, w3_ref, b3_ref,
                h_ref):
    xb = x_ref[...]
    h = jnp.dot(xb, w1_ref[...], preferred_element_type=jnp.float32) + b1_ref[...]
    h = jnp.maximum(h, 0.0)
    h = jnp.dot(h, w2_ref[...], preferred_element_type=jnp.float32) + b2_ref[...]
    h = jnp.maximum(h, 0.0)
    h = jnp.dot(h, w3_ref[...], preferred_element_type=jnp.float32) + b3_ref[...]
    h_ref[...] = h


def _embed(xp, W1, b1, W2, b2, W3, b3):
    grid = NP // QB1
    return pl.pallas_call(
        _embed_body,
        grid=(grid,),
        in_specs=[
            pl.BlockSpec((QB1, IN_DIM), lambda i: (i, 0)),
            pl.BlockSpec((IN_DIM, 128), lambda i: (0, 0)),
            pl.BlockSpec((1, 128), lambda i: (0, 0)),
            pl.BlockSpec((128, 64), lambda i: (0, 0)),
            pl.BlockSpec((1, 64), lambda i: (0, 0)),
            pl.BlockSpec((64, OUT_DIM), lambda i: (0, 0)),
            pl.BlockSpec((1, OUT_DIM), lambda i: (0, 0)),
        ],
        out_specs=pl.BlockSpec((QB1, OUT_DIM), lambda i: (i, 0)),
        out_shape=jax.ShapeDtypeStruct((NP, OUT_DIM), jnp.float32),
    )(xp, W1, b1.reshape(1, 128), W2, b2.reshape(1, 64), W3,
      b3.reshape(1, OUT_DIM))


def _knn_body(hq_ref, ht_ref, sqc_ref, sqr_ref, src_ref, dst_ref):
    i = pl.program_id(0)
    hq = hq_ref[...]
    ht = ht_ref[...]
    scores = jnp.dot(hq, ht, preferred_element_type=jnp.float32)
    sqq = sqc_ref[...]
    sqk = sqr_ref[...]
    d2 = sqq + sqk - 2.0 * scores
    neg = -d2
    kio = lax.broadcasted_iota(jnp.int32, (QB2, NP), 1)
    qid = i * QB2 + lax.broadcasted_iota(jnp.int32, (QB2, NP), 0)
    neg = jnp.where((kio == qid) | (kio >= N), -jnp.inf, neg)
    qrow = (i * QB2
            + lax.broadcasted_iota(jnp.int32, (1, QB2), 1))

    def body(j, carry):
        neg, m = carry
        pos = jnp.argmax(neg, axis=1).astype(jnp.int32)
        keep = (m > -4.0)
        posr = pos.reshape(1, QB2)
        keepr = keep.reshape(1, QB2)
        src_ref[pl.ds(j, 1), :] = jnp.where(keepr, posr, 0)
        dst_ref[pl.ds(j, 1), :] = jnp.where(keepr, qrow, 0)
        neg = jnp.where(kio == pos[:, None], -jnp.inf, neg)
        return neg, jnp.max(neg, axis=1, keepdims=True)

    m0 = jnp.max(neg, axis=1, keepdims=True)
    lax.fori_loop(0, K, body, (neg, m0))


def _knn(ht, h, sq_col, sq_row):
    grid = NP // QB2
    return pl.pallas_call(
        _knn_body,
        grid=(grid,),
        in_specs=[
            pl.BlockSpec((QB2, OUT_DIM), lambda i: (i, 0)),
            pl.BlockSpec((OUT_DIM, NP), lambda i: (0, 0)),
            pl.BlockSpec((QB2, 1), lambda i: (i, 0)),
            pl.BlockSpec((1, NP), lambda i: (0, 0)),
        ],
        out_specs=[
            pl.BlockSpec((K, QB2), lambda i: (0, i)),
            pl.BlockSpec((K, QB2), lambda i: (0, i)),
        ],
        out_shape=[
            jax.ShapeDtypeStruct((K, NP), jnp.int32),
            jax.ShapeDtypeStruct((K, NP), jnp.int32),
        ],
    )(h, ht, sq_col, sq_row)


def _sc_edges_body(src_h, tab_h, xs_h, idxs_v, rows_v, gsem):
    cid = lax.axis_index("c")
    sid = lax.axis_index("s")
    wid = sid * SC_CORES + cid

    def chunk_body(ci, _):
        c = wid + ci * NW

        @pl.when(c < NCHUNKS)
        def _():
            base = c * CHUNK
            pltpu.sync_copy(src_h.at[pl.ds(base, CHUNK)], idxs_v)
            pltpu.async_copy(tab_h.at[idxs_v], rows_v, gsem).wait()
            pltpu.sync_copy(rows_v, xs_h.at[pl.ds(base, CHUNK)])

        return 0

    lax.fori_loop(0, CHUNKS_PER_W, chunk_body, 0)


def _sc_edges(src_flat, tab):
    fn = pl.kernel(
        _sc_edges_body,
        out_type=jax.ShapeDtypeStruct((B, DT), jnp.float32),
        mesh=plsc.VectorSubcoreMesh(core_axis_name="c", subcore_axis_name="s",
                                    num_cores=SC_CORES,
                                    num_subcores=SC_SUBCORES),
        compiler_params=pltpu.CompilerParams(use_tc_tiling_on_sc=False),
        scratch_types=[
            pltpu.VMEM((CHUNK,), jnp.int32),
            pltpu.VMEM((CHUNK, DT), jnp.float32),
            pltpu.SemaphoreType.DMA,
        ],
    )
    return fn(src_flat, tab)


def _edge_attr_body(xs_ref, dst_ref, tab_ref, ea_ref, y_ref):
    i = pl.program_id(0)
    xs = xs_ref[...]
    dstc = dst_ref[...]
    xq = tab_ref[pl.ds(i * QB4, QB4), :]
    x0 = tab_ref[0:1, :]
    xdseq = jnp.broadcast_to(xq[:, None, :], (QB4, K, DT)).reshape(EB4, DT)
    m = dstc == 0
    xd = jnp.where(m, jnp.broadcast_to(x0, (EB4, DT)), xdseq)
    diff = xs - xd
    summ = xs + xd
    W = OUT_DIM + IN_DIM
    ea_ref[:, 0:W] = diff[:, 0:W]
    ea_ref[:, W:2 * W] = summ[:, 0:W]
    ys = xs[:, PID_COL:PID_COL + 1]
    yd = xd[:, PID_COL:PID_COL + 1]
    y_ref[...] = jnp.where(ys == yd, 1, 0).astype(jnp.int32)


def _edge_attr(xs_g, dst_col, tab):
    grid = N // QB4
    return pl.pallas_call(
        _edge_attr_body,
        grid=(grid,),
        in_specs=[
            pl.BlockSpec((EB4, DT), lambda i: (i, 0)),
            pl.BlockSpec((EB4, 1), lambda i: (i, 0)),
            pl.BlockSpec((NP, DT), lambda i: (0, 0)),
        ],
        out_specs=[
            pl.BlockSpec((EB4, 2 * (OUT_DIM + IN_DIM)), lambda i: (i, 0)),
            pl.BlockSpec((EB4, 1), lambda i: (i, 0)),
        ],
        out_shape=[
            jax.ShapeDtypeStruct((B, 2 * (OUT_DIM + IN_DIM)), jnp.float32),
            jax.ShapeDtypeStruct((B, 1), jnp.int32),
        ],
    )(xs_g, dst_col, tab)


def kernel(x, particle_id, pt, sector, reconstructable,
           W1, b1, W2, b2, W3, b3):
    xp = jnp.zeros((NP, IN_DIM), jnp.float32).at[:N].set(x)
    pidf = jnp.zeros((NP, 1), jnp.float32).at[:N, 0].set(
        particle_id.astype(jnp.float32))

    h3 = _embed(xp, W1, b1, W2, b2, W3, b3)
    n = jnp.linalg.norm(h3, axis=-1, keepdims=True)
    h = h3 / jnp.maximum(n, 1e-12)
    sq = jnp.sum(h * h, axis=1)
    ht = h.T
    tab = jnp.concatenate(
        [h, xp, jnp.zeros((NP, PID_COL - OUT_DIM - IN_DIM), jnp.float32),
         pidf, jnp.zeros((NP, DT - PID_COL - 1), jnp.float32)], axis=1)
    src_t, dst_t = _knn(ht, h, sq[:, None], sq[None, :])

    src_m = src_t[:, :N].T
    dst_m = dst_t[:, :N].T
    src_flat = src_m.reshape(B)
    dst_flat = dst_m.reshape(B)

    xs_g = _sc_edges(src_flat, tab)
    edge_attr, y_col = _edge_attr(xs_g, dst_flat.reshape(B, 1), tab)

    x_out = tab[:N, :OUT_DIM + IN_DIM]
    edge_index = jnp.stack([src_flat, dst_flat])
    return x_out, edge_index, y_col.reshape(B), edge_attr


# speedup vs baseline: 1.3342x; 1.3342x over previous
import jax
import jax.numpy as jnp
from jax import lax
from jax.experimental import pallas as pl
from jax.experimental.pallas import tpu as pltpu
from jax.experimental.pallas import tpu_sc as plsc

N = 10000
IN_DIM = 14
OUT_DIM = 8
K = 64
NP = 10240
DT = 32
PID_COL = 30
B = N * K

SC_CORES = 2
SC_SUBCORES = 16
NW = SC_CORES * SC_SUBCORES
CHUNK = 512
NCHUNKS = B // CHUNK
CHUNKS_PER_W = -(-NCHUNKS // NW)

QB1 = 512
QB2 = 256
QB4 = 40
EB4 = QB4 * K


def _embed_body(x_ref, w1_ref, b1_ref, w2_ref, b2_ref, w3_ref, b3_ref,
                h_ref):
    xb = x_ref[...]
    h = jnp.dot(xb, w1_ref[...], preferred_element_type=jnp.float32) + b1_ref[...]
    h = jnp.maximum(h, 0.0)
    h = jnp.dot(h, w2_ref[...], preferred_element_type=jnp.float32) + b2_ref[...]
    h = jnp.maximum(h, 0.0)
    h = jnp.dot(h, w3_ref[...], preferred_element_type=jnp.float32) + b3_ref[...]
    h_ref[...] = h


def _embed(xp, W1, b1, W2, b2, W3, b3):
    grid = NP // QB1
    return pl.pallas_call(
        _embed_body,
        grid=(grid,),
        in_specs=[
            pl.BlockSpec((QB1, IN_DIM), lambda i: (i, 0)),
            pl.BlockSpec((IN_DIM, 128), lambda i: (0, 0)),
            pl.BlockSpec((1, 128), lambda i: (0, 0)),
            pl.BlockSpec((128, 64), lambda i: (0, 0)),
            pl.BlockSpec((1, 64), lambda i: (0, 0)),
            pl.BlockSpec((64, OUT_DIM), lambda i: (0, 0)),
            pl.BlockSpec((1, OUT_DIM), lambda i: (0, 0)),
        ],
        out_specs=pl.BlockSpec((QB1, OUT_DIM), lambda i: (i, 0)),
        out_shape=jax.ShapeDtypeStruct((NP, OUT_DIM), jnp.float32),
    )(xp, W1, b1.reshape(1, 128), W2, b2.reshape(1, 64), W3,
      b3.reshape(1, OUT_DIM))


def _knn_body(hq_ref, ht_ref, sqc_ref, sqr_ref, src_ref, dst_ref):
    i = pl.program_id(0)
    hq = hq_ref[...]
    ht = ht_ref[...]
    scores = jnp.dot(hq, ht, preferred_element_type=jnp.float32)
    sqq = sqc_ref[...]
    sqk = sqr_ref[...]
    d2 = sqq + sqk - 2.0 * scores
    neg = -d2
    kio = lax.broadcasted_iota(jnp.int32, (QB2, NP), 1)
    qid = i * QB2 + lax.broadcasted_iota(jnp.int32, (QB2, NP), 0)
    neg = jnp.where((kio == qid) | (kio >= N), -jnp.inf, neg)
    qrow = (i * QB2
            + lax.broadcasted_iota(jnp.int32, (1, QB2), 1))

    def emit(j, m, pos):
        keep = (m > -4.0)
        posr = pos.reshape(1, QB2)
        keepr = keep.reshape(1, QB2)
        src_ref[pl.ds(j, 1), :] = jnp.where(keepr, posr, 0)
        dst_ref[pl.ds(j, 1), :] = jnp.where(keepr, qrow, 0)

    def body(j, carry):
        neg, m1 = carry
        pos1 = jnp.min(jnp.where(neg == m1, kio, NP), axis=1)
        emit(2 * j, m1, pos1)
        negx = jnp.where(kio == pos1[:, None], -jnp.inf, neg)
        m2 = jnp.max(negx, axis=1, keepdims=True)
        pos2 = jnp.min(jnp.where(negx == m2, kio, NP), axis=1)
        emit(2 * j + 1, m2, pos2)
        neg2 = jnp.where(kio == pos2[:, None], -jnp.inf, negx)
        return neg2, jnp.max(neg2, axis=1, keepdims=True)

    m0 = jnp.max(neg, axis=1, keepdims=True)
    lax.fori_loop(0, K // 2, body, (neg, m0))


def _knn(ht, h, sq_col, sq_row):
    grid = NP // QB2
    return pl.pallas_call(
        _knn_body,
        grid=(grid,),
        in_specs=[
            pl.BlockSpec((QB2, OUT_DIM), lambda i: (i, 0)),
            pl.BlockSpec((OUT_DIM, NP), lambda i: (0, 0)),
            pl.BlockSpec((QB2, 1), lambda i: (i, 0)),
            pl.BlockSpec((1, NP), lambda i: (0, 0)),
        ],
        out_specs=[
            pl.BlockSpec((K, QB2), lambda i: (0, i)),
            pl.BlockSpec((K, QB2), lambda i: (0, i)),
        ],
        out_shape=[
            jax.ShapeDtypeStruct((K, NP), jnp.int32),
            jax.ShapeDtypeStruct((K, NP), jnp.int32),
        ],
    )(h, ht, sq_col, sq_row)


def _sc_edges_body(src_h, tab_h, xs_h, idxs_v, rows_v, gsem):
    cid = lax.axis_index("c")
    sid = lax.axis_index("s")
    wid = sid * SC_CORES + cid

    def chunk_body(ci, _):
        c = wid + ci * NW

        @pl.when(c < NCHUNKS)
        def _():
            base = c * CHUNK
            pltpu.sync_copy(src_h.at[pl.ds(base, CHUNK)], idxs_v)
            pltpu.async_copy(tab_h.at[idxs_v], rows_v, gsem).wait()
            pltpu.sync_copy(rows_v, xs_h.at[pl.ds(base, CHUNK)])

        return 0

    lax.fori_loop(0, CHUNKS_PER_W, chunk_body, 0)


def _sc_edges(src_flat, tab):
    fn = pl.kernel(
        _sc_edges_body,
        out_type=jax.ShapeDtypeStruct((B, DT), jnp.float32),
        mesh=plsc.VectorSubcoreMesh(core_axis_name="c", subcore_axis_name="s",
                                    num_cores=SC_CORES,
                                    num_subcores=SC_SUBCORES),
        compiler_params=pltpu.CompilerParams(use_tc_tiling_on_sc=False),
        scratch_types=[
            pltpu.VMEM((CHUNK,), jnp.int32),
            pltpu.VMEM((CHUNK, DT), jnp.float32),
            pltpu.SemaphoreType.DMA,
        ],
    )
    return fn(src_flat, tab)


def _edge_attr_body(xs_ref, dst_ref, tab_ref, ea_ref, y_ref):
    i = pl.program_id(0)
    xs = xs_ref[...]
    dstc = dst_ref[...]
    xq = tab_ref[pl.ds(i * QB4, QB4), :]
    x0 = tab_ref[0:1, :]
    xdseq = jnp.broadcast_to(xq[:, None, :], (QB4, K, DT)).reshape(EB4, DT)
    m = dstc == 0
    xd = jnp.where(m, jnp.broadcast_to(x0, (EB4, DT)), xdseq)
    diff = xs - xd
    summ = xs + xd
    W = OUT_DIM + IN_DIM
    ea_ref[:, 0:W] = diff[:, 0:W]
    ea_ref[:, W:2 * W] = summ[:, 0:W]
    ys = xs[:, PID_COL:PID_COL + 1]
    yd = xd[:, PID_COL:PID_COL + 1]
    y_ref[...] = jnp.where(ys == yd, 1, 0).astype(jnp.int32)


def _edge_attr(xs_g, dst_col, tab):
    grid = N // QB4
    return pl.pallas_call(
        _edge_attr_body,
        grid=(grid,),
        in_specs=[
            pl.BlockSpec((EB4, DT), lambda i: (i, 0)),
            pl.BlockSpec((EB4, 1), lambda i: (i, 0)),
            pl.BlockSpec((NP, DT), lambda i: (0, 0)),
        ],
        out_specs=[
            pl.BlockSpec((EB4, 2 * (OUT_DIM + IN_DIM)), lambda i: (i, 0)),
            pl.BlockSpec((EB4, 1), lambda i: (i, 0)),
        ],
        out_shape=[
            jax.ShapeDtypeStruct((B, 2 * (OUT_DIM + IN_DIM)), jnp.float32),
            jax.ShapeDtypeStruct((B, 1), jnp.int32),
        ],
    )(xs_g, dst_col, tab)


def kernel(x, particle_id, pt, sector, reconstructable,
           W1, b1, W2, b2, W3, b3):
    xp = jnp.zeros((NP, IN_DIM), jnp.float32).at[:N].set(x)
    pidf = jnp.zeros((NP, 1), jnp.float32).at[:N, 0].set(
        particle_id.astype(jnp.float32))

    h3 = _embed(xp, W1, b1, W2, b2, W3, b3)
    n = jnp.linalg.norm(h3, axis=-1, keepdims=True)
    h = h3 / jnp.maximum(n, 1e-12)
    sq = jnp.sum(h * h, axis=1)
    ht = h.T
    tab = jnp.concatenate(
        [h, xp, jnp.zeros((NP, PID_COL - OUT_DIM - IN_DIM), jnp.float32),
         pidf, jnp.zeros((NP, DT - PID_COL - 1), jnp.float32)], axis=1)
    src_t, dst_t = _knn(ht, h, sq[:, None], sq[None, :])

    src_m = src_t[:, :N].T
    dst_m = dst_t[:, :N].T
    src_flat = src_m.reshape(B)
    dst_flat = dst_m.reshape(B)

    xs_g = _sc_edges(src_flat, tab)
    edge_attr, y_col = _edge_attr(xs_g, dst_flat.reshape(B, 1), tab)

    x_out = tab[:N, :OUT_DIM + IN_DIM]
    edge_index = jnp.stack([src_flat, dst_flat])
    return x_out, edge_index, y_col.reshape(B), edge_attr


# speedup vs baseline: 1.5100x; 1.1317x over previous
import jax
import jax.numpy as jnp
from jax import lax
from jax.experimental import pallas as pl
from jax.experimental.pallas import tpu as pltpu
from jax.experimental.pallas import tpu_sc as plsc

N = 10000
IN_DIM = 14
OUT_DIM = 8
K = 64
NP = 10240
DT = 32
PID_COL = 30
B = N * K

SC_CORES = 2
SC_SUBCORES = 16
NW = SC_CORES * SC_SUBCORES
CHUNK = 512
NCHUNKS = B // CHUNK
CHUNKS_PER_W = -(-NCHUNKS // NW)

QB1 = 512
QB2 = 256
QB4 = 40
EB4 = QB4 * K


def _embed_body(x_ref, w1_ref, b1_ref, w2_ref, b2_ref, w3_ref, b3_ref,
                h_ref):
    xb = x_ref[...]
    h = jnp.dot(xb, w1_ref[...], preferred_element_type=jnp.float32) + b1_ref[...]
    h = jnp.maximum(h, 0.0)
    h = jnp.dot(h, w2_ref[...], preferred_element_type=jnp.float32) + b2_ref[...]
    h = jnp.maximum(h, 0.0)
    h = jnp.dot(h, w3_ref[...], preferred_element_type=jnp.float32) + b3_ref[...]
    h_ref[...] = h


def _embed(xp, W1, b1, W2, b2, W3, b3):
    grid = NP // QB1
    return pl.pallas_call(
        _embed_body,
        grid=(grid,),
        in_specs=[
            pl.BlockSpec((QB1, IN_DIM), lambda i: (i, 0)),
            pl.BlockSpec((IN_DIM, 128), lambda i: (0, 0)),
            pl.BlockSpec((1, 128), lambda i: (0, 0)),
            pl.BlockSpec((128, 64), lambda i: (0, 0)),
            pl.BlockSpec((1, 64), lambda i: (0, 0)),
            pl.BlockSpec((64, OUT_DIM), lambda i: (0, 0)),
            pl.BlockSpec((1, OUT_DIM), lambda i: (0, 0)),
        ],
        out_specs=pl.BlockSpec((QB1, OUT_DIM), lambda i: (i, 0)),
        out_shape=jax.ShapeDtypeStruct((NP, OUT_DIM), jnp.float32),
    )(xp, W1, b1.reshape(1, 128), W2, b2.reshape(1, 64), W3,
      b3.reshape(1, OUT_DIM))


def _knn_body(hq_ref, ht_ref, sqc_ref, sqr_ref, src_ref, dst_ref):
    i = pl.program_id(0)
    hq = hq_ref[...]
    ht = ht_ref[...]
    scores = jnp.dot(hq, ht, preferred_element_type=jnp.float32)
    sqq = sqc_ref[...]
    sqk = sqr_ref[...]
    d2 = sqq + sqk - 2.0 * scores
    neg = -d2
    kio = lax.broadcasted_iota(jnp.int32, (QB2, NP), 1)
    qid = i * QB2 + lax.broadcasted_iota(jnp.int32, (QB2, NP), 0)
    neg = jnp.where((kio == qid) | (kio >= N), -jnp.inf, neg)
    qrow = (i * QB2
            + lax.broadcasted_iota(jnp.int32, (1, QB2), 1))

    def emit(j, m, pos):
        keep = (m > -4.0)
        posr = pos.reshape(1, QB2)
        keepr = keep.reshape(1, QB2)
        src_ref[pl.ds(j, 1), :] = jnp.where(keepr, posr, 0)
        dst_ref[pl.ds(j, 1), :] = jnp.where(keepr, qrow, 0)

    UNROLL = 4

    def body(j, carry):
        neg, m = carry
        for u in range(UNROLL):
            pos = jnp.min(jnp.where(neg == m, kio, NP), axis=1)
            emit(UNROLL * j + u, m, pos)
            neg = jnp.where(kio == pos[:, None], -jnp.inf, neg)
            m = jnp.max(neg, axis=1, keepdims=True)
        return neg, m

    m0 = jnp.max(neg, axis=1, keepdims=True)
    lax.fori_loop(0, K // UNROLL, body, (neg, m0))


def _knn(ht, h, sq_col, sq_row):
    grid = NP // QB2
    return pl.pallas_call(
        _knn_body,
        grid=(grid,),
        in_specs=[
            pl.BlockSpec((QB2, OUT_DIM), lambda i: (i, 0)),
            pl.BlockSpec((OUT_DIM, NP), lambda i: (0, 0)),
            pl.BlockSpec((QB2, 1), lambda i: (i, 0)),
            pl.BlockSpec((1, NP), lambda i: (0, 0)),
        ],
        out_specs=[
            pl.BlockSpec((K, QB2), lambda i: (0, i)),
            pl.BlockSpec((K, QB2), lambda i: (0, i)),
        ],
        out_shape=[
            jax.ShapeDtypeStruct((K, NP), jnp.int32),
            jax.ShapeDtypeStruct((K, NP), jnp.int32),
        ],
    )(h, ht, sq_col, sq_row)


def _sc_edges_body(src_h, tab_h, xs_h, idxs_v, rows_v, gsem):
    cid = lax.axis_index("c")
    sid = lax.axis_index("s")
    wid = sid * SC_CORES + cid

    def chunk_body(ci, _):
        c = wid + ci * NW

        @pl.when(c < NCHUNKS)
        def _():
            base = c * CHUNK
            pltpu.sync_copy(src_h.at[pl.ds(base, CHUNK)], idxs_v)
            pltpu.async_copy(tab_h.at[idxs_v], rows_v, gsem).wait()
            pltpu.sync_copy(rows_v, xs_h.at[pl.ds(base, CHUNK)])

        return 0

    lax.fori_loop(0, CHUNKS_PER_W, chunk_body, 0)


def _sc_edges(src_flat, tab):
    fn = pl.kernel(
        _sc_edges_body,
        out_type=jax.ShapeDtypeStruct((B, DT), jnp.float32),
        mesh=plsc.VectorSubcoreMesh(core_axis_name="c", subcore_axis_name="s",
                                    num_cores=SC_CORES,
                                    num_subcores=SC_SUBCORES),
        compiler_params=pltpu.CompilerParams(use_tc_tiling_on_sc=False),
        scratch_types=[
            pltpu.VMEM((CHUNK,), jnp.int32),
            pltpu.VMEM((CHUNK, DT), jnp.float32),
            pltpu.SemaphoreType.DMA,
        ],
    )
    return fn(src_flat, tab)


def _edge_attr_body(xs_ref, dst_ref, tab_ref, ea_ref, y_ref):
    i = pl.program_id(0)
    xs = xs_ref[...]
    dstc = dst_ref[...]
    xq = tab_ref[pl.ds(i * QB4, QB4), :]
    x0 = tab_ref[0:1, :]
    xdseq = jnp.broadcast_to(xq[:, None, :], (QB4, K, DT)).reshape(EB4, DT)
    m = dstc == 0
    xd = jnp.where(m, jnp.broadcast_to(x0, (EB4, DT)), xdseq)
    diff = xs - xd
    summ = xs + xd
    W = OUT_DIM + IN_DIM
    ea_ref[:, 0:W] = diff[:, 0:W]
    ea_ref[:, W:2 * W] = summ[:, 0:W]
    ys = xs[:, PID_COL:PID_COL + 1]
    yd = xd[:, PID_COL:PID_COL + 1]
    y_ref[...] = jnp.where(ys == yd, 1, 0).astype(jnp.int32)


def _edge_attr(xs_g, dst_col, tab):
    grid = N // QB4
    return pl.pallas_call(
        _edge_attr_body,
        grid=(grid,),
        in_specs=[
            pl.BlockSpec((EB4, DT), lambda i: (i, 0)),
            pl.BlockSpec((EB4, 1), lambda i: (i, 0)),
            pl.BlockSpec((NP, DT), lambda i: (0, 0)),
        ],
        out_specs=[
            pl.BlockSpec((EB4, 2 * (OUT_DIM + IN_DIM)), lambda i: (i, 0)),
            pl.BlockSpec((EB4, 1), lambda i: (i, 0)),
        ],
        out_shape=[
            jax.ShapeDtypeStruct((B, 2 * (OUT_DIM + IN_DIM)), jnp.float32),
            jax.ShapeDtypeStruct((B, 1), jnp.int32),
        ],
    )(xs_g, dst_col, tab)


def kernel(x, particle_id, pt, sector, reconstructable,
           W1, b1, W2, b2, W3, b3):
    xp = jnp.zeros((NP, IN_DIM), jnp.float32).at[:N].set(x)
    pidf = jnp.zeros((NP, 1), jnp.float32).at[:N, 0].set(
        particle_id.astype(jnp.float32))

    h3 = _embed(xp, W1, b1, W2, b2, W3, b3)
    n = jnp.linalg.norm(h3, axis=-1, keepdims=True)
    h = h3 / jnp.maximum(n, 1e-12)
    sq = jnp.sum(h * h, axis=1)
    ht = h.T
    tab = jnp.concatenate(
        [h, xp, jnp.zeros((NP, PID_COL - OUT_DIM - IN_DIM), jnp.float32),
         pidf, jnp.zeros((NP, DT - PID_COL - 1), jnp.float32)], axis=1)
    src_t, dst_t = _knn(ht, h, sq[:, None], sq[None, :])

    src_m = src_t[:, :N].T
    dst_m = dst_t[:, :N].T
    src_flat = src_m.reshape(B)
    dst_flat = dst_m.reshape(B)

    xs_g = _sc_edges(src_flat, tab)
    edge_attr, y_col = _edge_attr(xs_g, dst_flat.reshape(B, 1), tab)

    x_out = tab[:N, :OUT_DIM + IN_DIM]
    edge_index = jnp.stack([src_flat, dst_flat])
    return x_out, edge_index, y_col.reshape(B), edge_attr


# speedup vs baseline: 1.6084x; 1.0652x over previous
import jax
import jax.numpy as jnp
from jax import lax
from jax.experimental import pallas as pl
from jax.experimental.pallas import tpu as pltpu
from jax.experimental.pallas import tpu_sc as plsc

N = 10000
IN_DIM = 14
OUT_DIM = 8
K = 64
NP = 10240
DT = 32
PID_COL = 30
B = N * K

SC_CORES = 2
SC_SUBCORES = 16
NW = SC_CORES * SC_SUBCORES
CHUNK = 512
NCHUNKS = B // CHUNK
CHUNKS_PER_W = -(-NCHUNKS // NW)

QB1 = 512
QB2 = 256
QB4 = 40
EB4 = QB4 * K


def _embed_body(x_ref, w1_ref, b1_ref, w2_ref, b2_ref, w3_ref, b3_ref,
                h_ref):
    xb = x_ref[...]
    h = jnp.dot(xb, w1_ref[...], preferred_element_type=jnp.float32) + b1_ref[...]
    h = jnp.maximum(h, 0.0)
    h = jnp.dot(h, w2_ref[...], preferred_element_type=jnp.float32) + b2_ref[...]
    h = jnp.maximum(h, 0.0)
    h = jnp.dot(h, w3_ref[...], preferred_element_type=jnp.float32) + b3_ref[...]
    h_ref[...] = h


def _embed(xp, W1, b1, W2, b2, W3, b3):
    grid = NP // QB1
    return pl.pallas_call(
        _embed_body,
        grid=(grid,),
        in_specs=[
            pl.BlockSpec((QB1, IN_DIM), lambda i: (i, 0)),
            pl.BlockSpec((IN_DIM, 128), lambda i: (0, 0)),
            pl.BlockSpec((1, 128), lambda i: (0, 0)),
            pl.BlockSpec((128, 64), lambda i: (0, 0)),
            pl.BlockSpec((1, 64), lambda i: (0, 0)),
            pl.BlockSpec((64, OUT_DIM), lambda i: (0, 0)),
            pl.BlockSpec((1, OUT_DIM), lambda i: (0, 0)),
        ],
        out_specs=pl.BlockSpec((QB1, OUT_DIM), lambda i: (i, 0)),
        out_shape=jax.ShapeDtypeStruct((NP, OUT_DIM), jnp.float32),
    )(xp, W1, b1.reshape(1, 128), W2, b2.reshape(1, 64), W3,
      b3.reshape(1, OUT_DIM))


def _knn_body(hq_ref, ht_ref, sqc_ref, sqr_ref, src_ref, dst_ref):
    i = pl.program_id(0)
    hq = hq_ref[...]
    ht = ht_ref[...]
    scores = jnp.dot(hq, ht, preferred_element_type=jnp.float32)
    sqq = sqc_ref[...]
    sqk = sqr_ref[...]
    d2 = sqq + sqk - 2.0 * scores
    neg = -d2
    kio = lax.broadcasted_iota(jnp.int32, (QB2, NP), 1)
    qid = i * QB2 + lax.broadcasted_iota(jnp.int32, (QB2, NP), 0)
    neg = jnp.where((kio == qid) | (kio >= N), -jnp.inf, neg)
    qrow = (i * QB2
            + lax.broadcasted_iota(jnp.int32, (1, QB2), 1))

    def emit(j, m, pos):
        keep = (m > -4.0)
        posr = pos.reshape(1, QB2)
        keepr = keep.reshape(1, QB2)
        src_ref[pl.ds(j, 1), :] = jnp.where(keepr, posr, 0)
        dst_ref[pl.ds(j, 1), :] = jnp.where(keepr, qrow, 0)

    UNROLL = 8

    def body(j, carry):
        neg, m = carry
        for u in range(UNROLL):
            pos = jnp.min(jnp.where(neg == m, kio, NP), axis=1)
            emit(UNROLL * j + u, m, pos)
            neg = jnp.where(kio == pos[:, None], -jnp.inf, neg)
            m = jnp.max(neg, axis=1, keepdims=True)
        return neg, m

    m0 = jnp.max(neg, axis=1, keepdims=True)
    lax.fori_loop(0, K // UNROLL, body, (neg, m0))


def _knn(ht, h, sq_col, sq_row):
    grid = NP // QB2
    return pl.pallas_call(
        _knn_body,
        grid=(grid,),
        in_specs=[
            pl.BlockSpec((QB2, OUT_DIM), lambda i: (i, 0)),
            pl.BlockSpec((OUT_DIM, NP), lambda i: (0, 0)),
            pl.BlockSpec((QB2, 1), lambda i: (i, 0)),
            pl.BlockSpec((1, NP), lambda i: (0, 0)),
        ],
        out_specs=[
            pl.BlockSpec((K, QB2), lambda i: (0, i)),
            pl.BlockSpec((K, QB2), lambda i: (0, i)),
        ],
        out_shape=[
            jax.ShapeDtypeStruct((K, NP), jnp.int32),
            jax.ShapeDtypeStruct((K, NP), jnp.int32),
        ],
    )(h, ht, sq_col, sq_row)


def _sc_edges_body(src_h, tab_h, xs_h, idxs_v, rows_v, gsem):
    cid = lax.axis_index("c")
    sid = lax.axis_index("s")
    wid = sid * SC_CORES + cid

    def chunk_body(ci, _):
        c = wid + ci * NW

        @pl.when(c < NCHUNKS)
        def _():
            base = c * CHUNK
            pltpu.sync_copy(src_h.at[pl.ds(base, CHUNK)], idxs_v)
            pltpu.async_copy(tab_h.at[idxs_v], rows_v, gsem).wait()
            pltpu.sync_copy(rows_v, xs_h.at[pl.ds(base, CHUNK)])

        return 0

    lax.fori_loop(0, CHUNKS_PER_W, chunk_body, 0)


def _sc_edges(src_flat, tab):
    fn = pl.kernel(
        _sc_edges_body,
        out_type=jax.ShapeDtypeStruct((B, DT), jnp.float32),
        mesh=plsc.VectorSubcoreMesh(core_axis_name="c", subcore_axis_name="s",
                                    num_cores=SC_CORES,
                                    num_subcores=SC_SUBCORES),
        compiler_params=pltpu.CompilerParams(use_tc_tiling_on_sc=False),
        scratch_types=[
            pltpu.VMEM((CHUNK,), jnp.int32),
            pltpu.VMEM((CHUNK, DT), jnp.float32),
            pltpu.SemaphoreType.DMA,
        ],
    )
    return fn(src_flat, tab)


def _edge_attr_body(xs_ref, dst_ref, tab_ref, ea_ref, y_ref):
    i = pl.program_id(0)
    xs = xs_ref[...]
    dstc = dst_ref[...]
    xq = tab_ref[pl.ds(i * QB4, QB4), :]
    x0 = tab_ref[0:1, :]
    xdseq = jnp.broadcast_to(xq[:, None, :], (QB4, K, DT)).reshape(EB4, DT)
    m = dstc == 0
    xd = jnp.where(m, jnp.broadcast_to(x0, (EB4, DT)), xdseq)
    diff = xs - xd
    summ = xs + xd
    W = OUT_DIM + IN_DIM
    ea_ref[:, 0:W] = diff[:, 0:W]
    ea_ref[:, W:2 * W] = summ[:, 0:W]
    ys = xs[:, PID_COL:PID_COL + 1]
    yd = xd[:, PID_COL:PID_COL + 1]
    y_ref[...] = jnp.where(ys == yd, 1, 0).astype(jnp.int32)


def _edge_attr(xs_g, dst_col, tab):
    grid = N // QB4
    return pl.pallas_call(
        _edge_attr_body,
        grid=(grid,),
        in_specs=[
            pl.BlockSpec((EB4, DT), lambda i: (i, 0)),
            pl.BlockSpec((EB4, 1), lambda i: (i, 0)),
            pl.BlockSpec((NP, DT), lambda i: (0, 0)),
        ],
        out_specs=[
            pl.BlockSpec((EB4, 2 * (OUT_DIM + IN_DIM)), lambda i: (i, 0)),
            pl.BlockSpec((EB4, 1), lambda i: (i, 0)),
        ],
        out_shape=[
            jax.ShapeDtypeStruct((B, 2 * (OUT_DIM + IN_DIM)), jnp.float32),
            jax.ShapeDtypeStruct((B, 1), jnp.int32),
        ],
    )(xs_g, dst_col, tab)


def kernel(x, particle_id, pt, sector, reconstructable,
           W1, b1, W2, b2, W3, b3):
    xp = jnp.zeros((NP, IN_DIM), jnp.float32).at[:N].set(x)
    pidf = jnp.zeros((NP, 1), jnp.float32).at[:N, 0].set(
        particle_id.astype(jnp.float32))

    h3 = _embed(xp, W1, b1, W2, b2, W3, b3)
    n = jnp.linalg.norm(h3, axis=-1, keepdims=True)
    h = h3 / jnp.maximum(n, 1e-12)
    sq = jnp.sum(h * h, axis=1)
    ht = h.T
    tab = jnp.concatenate(
        [h, xp, jnp.zeros((NP, PID_COL - OUT_DIM - IN_DIM), jnp.float32),
         pidf, jnp.zeros((NP, DT - PID_COL - 1), jnp.float32)], axis=1)
    src_t, dst_t = _knn(ht, h, sq[:, None], sq[None, :])

    src_m = src_t[:, :N].T
    dst_m = dst_t[:, :N].T
    src_flat = src_m.reshape(B)
    dst_flat = dst_m.reshape(B)

    xs_g = _sc_edges(src_flat, tab)
    edge_attr, y_col = _edge_attr(xs_g, dst_flat.reshape(B, 1), tab)

    x_out = tab[:N, :OUT_DIM + IN_DIM]
    edge_index = jnp.stack([src_flat, dst_flat])
    return x_out, edge_index, y_col.reshape(B), edge_attr
